# baseline (speedup 1.0000x reference)
import functools

import jax
import jax.numpy as jnp
from jax import lax
from jax.experimental import pallas as pl
from jax.experimental.pallas import tpu as pltpu
from jax.experimental.pallas import tpu_sc as plsc

_L = 16
_R = 8

_mesh = plsc.VectorSubcoreMesh(core_axis_name="c", subcore_axis_name="s")


@functools.partial(
    pl.kernel,
    mesh=_mesh,
    out_type=jax.ShapeDtypeStruct((_R, _L), jnp.float32),
    scratch_types=[
        pltpu.VMEM((_L,), jnp.float32),
        pltpu.VMEM((_R, _L), jnp.float32),
        pltpu.VMEM((_R, _L), jnp.float32),
    ],
)
def _sc_broadcast_mul(a_hbm, b_hbm, out_hbm, a_v, b_v, o_v):
    c = lax.axis_index("c")
    s = lax.axis_index("s")

    @pl.when(jnp.logical_and(c == 0, s == 0))
    def _():
        pltpu.sync_copy(a_hbm, a_v)
        pltpu.sync_copy(b_hbm, b_v)
        av = a_v[...]
        for r in range(_R):
            o_v[r, :] = av[r] * b_v[r, :]
        pltpu.sync_copy(o_v, out_hbm)


def kernel(x, a_dense, b_dense):
    del x
    a_flat = jnp.pad(a_dense.reshape(-1), (0, _L - _R))
    b_rows = jnp.pad(b_dense.reshape(_R, -1), ((0, 0), (0, _L - 10)))
    out = _sc_broadcast_mul(a_flat, b_rows)
    return out[:, :10].reshape(b_dense.shape)


# speedup vs baseline: 1.1204x; 1.1204x over previous
import functools

import jax
import jax.numpy as jnp
from jax.experimental import pallas as pl
from jax.experimental.pallas import tpu as pltpu
from jax.experimental.pallas import tpu_sc as plsc

_L = 16
_R = 8

_mesh = plsc.VectorSubcoreMesh(
    core_axis_name="c", subcore_axis_name="s", num_cores=1, num_subcores=1
)


@functools.partial(
    pl.kernel,
    mesh=_mesh,
    out_type=jax.ShapeDtypeStruct((_R, _L), jnp.float32),
    scratch_types=[
        pltpu.VMEM((_L,), jnp.float32),
        pltpu.VMEM((_R, _L), jnp.float32),
        pltpu.VMEM((_R, _L), jnp.float32),
        pltpu.SemaphoreType.DMA,
        pltpu.SemaphoreType.DMA,
    ],
)
def _sc_broadcast_mul(a_hbm, b_hbm, out_hbm, a_v, b_v, o_v, sem_a, sem_b):
    cp_a = pltpu.make_async_copy(a_hbm, a_v, sem_a)
    cp_b = pltpu.make_async_copy(b_hbm, b_v, sem_b)
    cp_a.start()
    cp_b.start()
    cp_a.wait()
    cp_b.wait()
    av = a_v[...]
    for r in range(_R):
        o_v[r, :] = av[r] * b_v[r, :]
    pltpu.sync_copy(o_v, out_hbm)


def kernel(x, a_dense, b_dense):
    del x
    a_flat = jnp.pad(a_dense.reshape(-1), (0, _L - _R))
    b_rows = jnp.pad(b_dense.reshape(_R, -1), ((0, 0), (0, _L - 10)))
    out = _sc_broadcast_mul(a_flat, b_rows)
    return out[:, :10].reshape(b_dense.shape)
